# Initial kernel scaffold; baseline (speedup 1.0000x reference)
import jax
import jax.numpy as jnp
from jax.experimental import pallas as pl


def kernel(probas, labels):
    raise NotImplementedError("write your pallas kernel here")


# speedup vs baseline: 32.0882x; 32.0882x over previous
import functools

import jax
import jax.numpy as jnp
from jax import lax
from jax.experimental import pallas as pl
from jax.experimental.pallas import tpu as pltpu
from jax.experimental.pallas import tpu_sc as plsc

K = 4096
NCLS = 19
SUB = 65536
NSUB = 152
PIECE = 16384
NC, NS = 2, 16
NW = NC * NS
HROWS = 2 * K // 128
SROWS = 1280

_mesh = plsc.VectorSubcoreMesh(
    core_axis_name="c", subcore_axis_name="s", num_cores=NC, num_subcores=NS
)
_STRIPE = SROWS // NS


@functools.partial(
    pl.kernel,
    mesh=_mesh,
    compiler_params=pltpu.CompilerParams(needs_layout_passes=False),
    out_type=jax.ShapeDtypeStruct((NC, SROWS, 128), jnp.float32),
    scratch_types=[
        pltpu.VMEM((PIECE,), jnp.float32),
        pltpu.VMEM((PIECE,), jnp.int32),
        pltpu.VMEM((HROWS, 128), jnp.float32),
        pltpu.VMEM((HROWS,), jnp.int32),
        pltpu.VMEM_SHARED((SROWS, 128), jnp.float32),
    ],
)
def _hist_kernel(p_hbm, l_hbm, z_hbm, out_hbm, pbuf, lbuf, hist, idx, shared):
    cid = lax.axis_index("c")
    sid = lax.axis_index("s")
    wid = sid * NC + cid

    pltpu.sync_copy(
        z_hbm.at[pl.ds(sid * _STRIPE, _STRIPE)],
        shared.at[pl.ds(sid * _STRIPE, _STRIPE)],
    )
    pltpu.sync_copy(z_hbm.at[pl.ds(0, HROWS)], hist)
    plsc.subcore_barrier()

    ones = jnp.full((16,), 1.0, jnp.float32)
    it16 = lax.iota(jnp.int32, 16)

    def flush(c):
        base = c * HROWS
        for jj in range(HROWS // 16):
            idx[pl.ds(jj * 16, 16)] = base + jj * 16 + it16
        pltpu.sync_copy(hist, shared.at[idx], add=True)
        pltpu.sync_copy(z_hbm.at[pl.ds(0, HROWS)], hist)

    lo = (NSUB * wid) // NW
    hi = (NSUB * (wid + 1)) // NW

    def do_piece(base_elem):
        pltpu.sync_copy(p_hbm.at[pl.ds(base_elem, PIECE)], pbuf)
        pltpu.sync_copy(l_hbm.at[pl.ds(base_elem, PIECE)], lbuf)

        def inner(i, carry):
            for jj in range(8):
                off = i * 128 + jj * 16
                p = pbuf[pl.ds(off, 16)]
                lbl = lbuf[pl.ds(off, 16)]
                fgm = lbl > 0
                e = jnp.where(fgm, 1.0 - p, p)
                ki = (e * float(K)).astype(jnp.int32)
                ki = jnp.minimum(jnp.maximum(ki, 0), K - 1)
                row = lax.shift_right_logical(ki, 7)
                col = lax.bitwise_and(ki, 127)
                plsc.addupdate_scatter(hist, [row, col], ones)
                plsc.addupdate_scatter(
                    hist, [row + (K // 128), col], ones, mask=fgm
                )
            return carry

        lax.fori_loop(0, PIECE // 128, inner, 0)

    def body(rr, cur_c):
        c_rr = (rr % (NSUB // 2)) // 4

        @pl.when(c_rr != cur_c)
        def _():
            flush(cur_c)

        for q in range(SUB // PIECE):
            do_piece(rr * SUB + q * PIECE)
        return c_rr

    c0 = (lo % (NSUB // 2)) // 4
    last_c = lax.fori_loop(lo, hi, body, c0)
    flush(last_c)
    plsc.subcore_barrier()
    pltpu.sync_copy(
        shared.at[pl.ds(sid * _STRIPE, _STRIPE)],
        out_hbm.at[cid, pl.ds(sid * _STRIPE, _STRIPE)],
    )


def _combine_kernel(h_ref, out_ref):
    h = h_ref[0] + h_ref[1]
    R = HROWS // 2

    l0 = lax.broadcasted_iota(jnp.int32, (128, 128), 0)
    l1 = lax.broadcasted_iota(jnp.int32, (128, 128), 1)
    A = (l0 >= l1).astype(jnp.float32)
    r0 = lax.broadcasted_iota(jnp.int32, (R, R), 0)
    r1 = lax.broadcasted_iota(jnp.int32, (R, R), 1)
    B = (r1 > r0).astype(jnp.float32)

    total = jnp.float32(0.0)
    for c in range(NCLS):
        cls = h[c * HROWS:(c + 1) * HROWS]
        cnt = cls[:R]
        fgc = cls[R:]
        nl = jax.lax.dot(cnt, A, precision=jax.lax.Precision.HIGHEST)
        sl = jax.lax.dot(fgc, A, precision=jax.lax.Precision.HIGHEST)
        rs_n = jnp.sum(cnt, axis=1, keepdims=True)
        rs_s = jnp.sum(fgc, axis=1, keepdims=True)
        rse_n = jax.lax.dot(B, rs_n, precision=jax.lax.Precision.HIGHEST)
        rse_s = jax.lax.dot(B, rs_s, precision=jax.lax.Precision.HIGHEST)
        n = nl + rse_n
        s = sl + rse_s
        G = jnp.sum(fgc)
        J = n / jnp.maximum(G + n - s, 1e-30)
        total = total + (jnp.sum(J) - 0.5) / float(K)
    out_ref[0, 0] = total / float(NCLS)


def kernel(probas, labels):
    pf = probas.reshape(-1)
    lf = labels.reshape(-1)
    zeros = jnp.zeros((SROWS, 128), jnp.float32)
    hist = _hist_kernel(pf, lf, zeros)
    out = pl.pallas_call(
        _combine_kernel,
        out_shape=jax.ShapeDtypeStruct((1, 1), jnp.float32),
        out_specs=pl.BlockSpec(memory_space=pltpu.SMEM),
    )(hist)
    return out[0, 0]


# speedup vs baseline: 37.2110x; 1.1596x over previous
import functools

import jax
import jax.numpy as jnp
from jax import lax
from jax.experimental import pallas as pl
from jax.experimental.pallas import tpu as pltpu
from jax.experimental.pallas import tpu_sc as plsc

K = 4096
NCLS = 19
PIECE = 16384
NPIECE = 608
NC, NS = 2, 16
NW = NC * NS
PPW = NPIECE // NW
HROWS = 2 * K // 128
SROWS = 1280

_mesh = plsc.VectorSubcoreMesh(
    core_axis_name="c", subcore_axis_name="s", num_cores=NC, num_subcores=NS
)
_STRIPE = SROWS // NS


@functools.partial(
    pl.kernel,
    mesh=_mesh,
    compiler_params=pltpu.CompilerParams(needs_layout_passes=False),
    out_type=jax.ShapeDtypeStruct((NC, SROWS, 128), jnp.float32),
    scratch_types=[
        pltpu.VMEM((PIECE,), jnp.float32),
        pltpu.VMEM((PIECE,), jnp.float32),
        pltpu.VMEM((PIECE,), jnp.int32),
        pltpu.VMEM((PIECE,), jnp.int32),
        pltpu.VMEM((HROWS, 128), jnp.float32),
        pltpu.VMEM((HROWS,), jnp.int32),
        pltpu.VMEM_SHARED((SROWS, 128), jnp.float32),
        pltpu.SemaphoreType.DMA,
        pltpu.SemaphoreType.DMA,
        pltpu.SemaphoreType.DMA,
        pltpu.SemaphoreType.DMA,
    ],
)
def _hist_kernel(p_hbm, l_hbm, z_hbm, out_hbm,
                 pbuf0, pbuf1, lbuf0, lbuf1, hist, idx, shared,
                 sp0, sp1, sl0, sl1):
    cid = lax.axis_index("c")
    sid = lax.axis_index("s")
    wid = sid * NC + cid

    pbufs = (pbuf0, pbuf1)
    lbufs = (lbuf0, lbuf1)
    psems = (sp0, sp1)
    lsems = (sl0, sl1)

    def issue(i):
        base = (wid * PPW + i) * PIECE
        slot = i % 2
        hp = pltpu.async_copy(
            p_hbm.at[pl.ds(base, PIECE)], pbufs[slot], psems[slot])
        hl = pltpu.async_copy(
            l_hbm.at[pl.ds(base, PIECE)], lbufs[slot], lsems[slot])
        return hp, hl

    handles = issue(0)
    pltpu.sync_copy(
        z_hbm.at[pl.ds(sid * _STRIPE, _STRIPE)],
        shared.at[pl.ds(sid * _STRIPE, _STRIPE)],
    )
    pltpu.sync_copy(z_hbm.at[pl.ds(0, HROWS)], hist)
    plsc.subcore_barrier()

    ones = jnp.full((16,), 1.0, jnp.float32)
    it16 = lax.iota(jnp.int32, 16)

    def flush(c):
        base = c * HROWS
        for jj in range(HROWS // 16):
            idx[pl.ds(jj * 16, 16)] = base + jj * 16 + it16
        pltpu.sync_copy(hist, shared.at[idx], add=True)
        pltpu.sync_copy(z_hbm.at[pl.ds(0, HROWS)], hist)

    def piece_class(i):
        pi = wid * PPW + i
        return ((pi // 4) % 76) // 4

    cur_c = piece_class(0)
    for i in range(PPW):
        if i + 1 < PPW:
            next_handles = issue(i + 1)
        handles[0].wait()
        handles[1].wait()
        if i > 0:
            c_i = piece_class(i)

            @pl.when(c_i != cur_c)
            def _():
                flush(cur_c)

            cur_c = c_i
        pbuf = pbufs[i % 2]
        lbuf = lbufs[i % 2]

        def inner(j, carry, pbuf=pbuf, lbuf=lbuf):
            for jj in range(8):
                off = j * 128 + jj * 16
                p = pbuf[pl.ds(off, 16)]
                lbl = lbuf[pl.ds(off, 16)]
                fgm = lbl > 0
                e = jnp.where(fgm, 1.0 - p, p)
                ki = (e * float(K)).astype(jnp.int32)
                ki = jnp.minimum(ki, K - 1)
                k2 = ki + ki + lbl
                row = lax.shift_right_logical(k2, 7)
                col = lax.bitwise_and(k2, 127)
                plsc.addupdate_scatter(hist, [row, col], ones)
            return carry

        lax.fori_loop(0, PIECE // 128, inner, 0)
        if i + 1 < PPW:
            handles = next_handles
    flush(cur_c)
    plsc.subcore_barrier()
    pltpu.sync_copy(
        shared.at[pl.ds(sid * _STRIPE, _STRIPE)],
        out_hbm.at[cid, pl.ds(sid * _STRIPE, _STRIPE)],
    )


def _combine_kernel(h_ref, out_ref):
    h = h_ref[0] + h_ref[1]

    l0 = lax.broadcasted_iota(jnp.int32, (128, 128), 0)
    l1 = lax.broadcasted_iota(jnp.int32, (128, 128), 1)
    A = (l0 >= l1).astype(jnp.float32)
    r0 = lax.broadcasted_iota(jnp.int32, (HROWS, HROWS), 0)
    r1 = lax.broadcasted_iota(jnp.int32, (HROWS, HROWS), 1)
    B = (r1 > r0).astype(jnp.float32)
    lane = lax.broadcasted_iota(jnp.int32, (HROWS, 128), 1)
    odd = (lane % 2 == 1).astype(jnp.float32)
    even = 1.0 - odd

    total = jnp.float32(0.0)
    for c in range(NCLS):
        x = h[c * HROWS:(c + 1) * HROWS]
        xo = x * odd
        nl = jax.lax.dot(x, A, precision=jax.lax.Precision.HIGHEST)
        sl = jax.lax.dot(xo, A, precision=jax.lax.Precision.HIGHEST)
        rs_n = jnp.sum(x, axis=1, keepdims=True)
        rs_s = jnp.sum(xo, axis=1, keepdims=True)
        rse_n = jax.lax.dot(B, rs_n, precision=jax.lax.Precision.HIGHEST)
        rse_s = jax.lax.dot(B, rs_s, precision=jax.lax.Precision.HIGHEST)
        n = nl + rse_n
        s = sl + rse_s
        G = jnp.sum(xo)
        J = n / jnp.maximum(G + n - s, 1e-30)
        total = total + (jnp.sum(J * even) - 0.5) / float(K)
    out_ref[0, 0] = total / float(NCLS)


def kernel(probas, labels):
    pf = probas.reshape(-1)
    lf = labels.reshape(-1)
    zeros = jnp.zeros((SROWS, 128), jnp.float32)
    hist = _hist_kernel(pf, lf, zeros)
    out = pl.pallas_call(
        _combine_kernel,
        out_shape=jax.ShapeDtypeStruct((1, 1), jnp.float32),
        out_specs=pl.BlockSpec(memory_space=pltpu.SMEM),
    )(hist)
    return out[0, 0]


# speedup vs baseline: 92.5048x; 2.4860x over previous
import functools

import jax
import jax.numpy as jnp
from jax import lax
from jax.experimental import pallas as pl
from jax.experimental.pallas import tpu as pltpu
from jax.experimental.pallas import tpu_sc as plsc

K = 4096
KS = K * (1.0 - 1e-6)
NCLS = 19
PIECE = 16384
NPIECE = 608
NC, NS = 2, 16
NW = NC * NS
PPW = NPIECE // NW
HROWS = 2 * K // 128
SROWS = 1280

_mesh = plsc.VectorSubcoreMesh(
    core_axis_name="c", subcore_axis_name="s", num_cores=NC, num_subcores=NS
)
_STRIPE = SROWS // NS


@functools.partial(
    pl.kernel,
    mesh=_mesh,
    compiler_params=pltpu.CompilerParams(needs_layout_passes=False),
    out_type=jax.ShapeDtypeStruct((NC, SROWS, 128), jnp.float32),
    scratch_types=[
        pltpu.VMEM((PIECE,), jnp.float32),
        pltpu.VMEM((PIECE,), jnp.float32),
        pltpu.VMEM((PIECE,), jnp.int32),
        pltpu.VMEM((PIECE,), jnp.int32),
        pltpu.VMEM((HROWS, 128), jnp.float32),
        pltpu.VMEM((HROWS,), jnp.int32),
        pltpu.VMEM_SHARED((SROWS, 128), jnp.float32),
        pltpu.SemaphoreType.DMA,
        pltpu.SemaphoreType.DMA,
        pltpu.SemaphoreType.DMA,
        pltpu.SemaphoreType.DMA,
    ],
)
def _hist_kernel(p_hbm, l_hbm, z_hbm, out_hbm,
                 pbuf0, pbuf1, lbuf0, lbuf1, hist, idx, shared,
                 sp0, sp1, sl0, sl1):
    cid = lax.axis_index("c")
    sid = lax.axis_index("s")
    wid = sid * NC + cid

    pbufs = (pbuf0, pbuf1)
    lbufs = (lbuf0, lbuf1)
    psems = (sp0, sp1)
    lsems = (sl0, sl1)

    def issue(i):
        base = (wid * PPW + i) * PIECE
        slot = i % 2
        hp = pltpu.async_copy(
            p_hbm.at[pl.ds(base, PIECE)], pbufs[slot], psems[slot])
        hl = pltpu.async_copy(
            l_hbm.at[pl.ds(base, PIECE)], lbufs[slot], lsems[slot])
        return hp, hl

    handles = issue(0)
    pltpu.sync_copy(
        z_hbm.at[pl.ds(sid * _STRIPE, _STRIPE)],
        shared.at[pl.ds(sid * _STRIPE, _STRIPE)],
    )
    pltpu.sync_copy(z_hbm.at[pl.ds(0, HROWS)], hist)
    plsc.subcore_barrier()

    ones = jnp.full((16,), 1.0, jnp.float32)
    it16 = lax.iota(jnp.int32, 16)

    def flush(c):
        base = c * HROWS
        for jj in range(HROWS // 16):
            idx[pl.ds(jj * 16, 16)] = base + jj * 16 + it16
        pltpu.sync_copy(hist, shared.at[idx], add=True)
        pltpu.sync_copy(z_hbm.at[pl.ds(0, HROWS)], hist)

    def piece_class(i):
        pi = wid * PPW + i
        return ((pi // 4) % 76) // 4

    cur_c = piece_class(0)
    for i in range(PPW):
        if i + 1 < PPW:
            next_handles = issue(i + 1)
        handles[0].wait()
        handles[1].wait()
        if i > 0:
            c_i = piece_class(i)

            @pl.when(c_i != cur_c)
            def _():
                flush(cur_c)

            cur_c = c_i
        pbuf = pbufs[i % 2]
        lbuf = lbufs[i % 2]

        def vec_body(v, pbuf=pbuf, lbuf=lbuf):
            off = v * 16
            p = pbuf[pl.ds(off, 16)]
            lbl = lbuf[pl.ds(off, 16)]
            fgm = lbl > 0
            e = jnp.where(fgm, 1.0 - p, p)
            ki = (e * KS).astype(jnp.int32)
            k2 = ki + ki + lbl
            row = lax.shift_right_logical(k2, 7)
            col = lax.bitwise_and(k2, 127)
            plsc.addupdate_scatter(hist, [row, col], ones)

        plsc.parallel_loop(0, PIECE // 16, unroll=8)(vec_body)
        if i + 1 < PPW:
            handles = next_handles
    flush(cur_c)
    plsc.subcore_barrier()
    pltpu.sync_copy(
        shared.at[pl.ds(sid * _STRIPE, _STRIPE)],
        out_hbm.at[cid, pl.ds(sid * _STRIPE, _STRIPE)],
    )


def _combine_kernel(h_ref, out_ref):
    h = h_ref[0] + h_ref[1]

    l0 = lax.broadcasted_iota(jnp.int32, (128, 128), 0)
    l1 = lax.broadcasted_iota(jnp.int32, (128, 128), 1)
    A = (l0 >= l1).astype(jnp.float32)
    r0 = lax.broadcasted_iota(jnp.int32, (HROWS, HROWS), 0)
    r1 = lax.broadcasted_iota(jnp.int32, (HROWS, HROWS), 1)
    B = (r1 > r0).astype(jnp.float32)
    lane = lax.broadcasted_iota(jnp.int32, (HROWS, 128), 1)
    odd = (lane % 2 == 1).astype(jnp.float32)
    even = 1.0 - odd

    total = jnp.float32(0.0)
    for c in range(NCLS):
        x = h[c * HROWS:(c + 1) * HROWS]
        xo = x * odd
        nl = jax.lax.dot(x, A, precision=jax.lax.Precision.HIGHEST)
        sl = jax.lax.dot(xo, A, precision=jax.lax.Precision.HIGHEST)
        rs_n = jnp.sum(x, axis=1, keepdims=True)
        rs_s = jnp.sum(xo, axis=1, keepdims=True)
        rse_n = jax.lax.dot(B, rs_n, precision=jax.lax.Precision.HIGHEST)
        rse_s = jax.lax.dot(B, rs_s, precision=jax.lax.Precision.HIGHEST)
        n = nl + rse_n
        s = sl + rse_s
        G = jnp.sum(xo)
        J = n / jnp.maximum(G + n - s, 1e-30)
        total = total + (jnp.sum(J * even) - 0.5) / KS
    out_ref[0, 0] = total / float(NCLS)


def kernel(probas, labels):
    pf = probas.reshape(-1)
    lf = labels.reshape(-1)
    zeros = jnp.zeros((SROWS, 128), jnp.float32)
    hist = _hist_kernel(pf, lf, zeros)
    out = pl.pallas_call(
        _combine_kernel,
        out_shape=jax.ShapeDtypeStruct((1, 1), jnp.float32),
        out_specs=pl.BlockSpec(memory_space=pltpu.SMEM),
    )(hist)
    return out[0, 0]


# speedup vs baseline: 158.8302x; 1.7170x over previous
import functools

import jax
import jax.numpy as jnp
from jax import lax
from jax.experimental import pallas as pl
from jax.experimental.pallas import tpu as pltpu
from jax.experimental.pallas import tpu_sc as plsc

K = 4096
KS = K * (1.0 - 1e-6)
NCLS = 19
PIECE = 16384
NPIECE = 608
NC, NS = 2, 16
NW = NC * NS
PPW = NPIECE // NW
HROWS = 2 * K // 128
SROWS = 1280

_mesh = plsc.VectorSubcoreMesh(
    core_axis_name="c", subcore_axis_name="s", num_cores=NC, num_subcores=NS
)
_STRIPE = SROWS // NS


@functools.partial(
    pl.kernel,
    mesh=_mesh,
    compiler_params=pltpu.CompilerParams(needs_layout_passes=False),
    out_type=jax.ShapeDtypeStruct((NC, SROWS, 128), jnp.float32),
    scratch_types=[
        pltpu.VMEM((PIECE // 256, 256), jnp.float32),
        pltpu.VMEM((PIECE // 256, 256), jnp.float32),
        pltpu.VMEM((PIECE // 256, 256), jnp.int32),
        pltpu.VMEM((PIECE // 256, 256), jnp.int32),
        pltpu.VMEM((HROWS, 128), jnp.float32),
        pltpu.VMEM((HROWS,), jnp.int32),
        pltpu.VMEM_SHARED((SROWS, 128), jnp.float32),
        pltpu.SemaphoreType.DMA,
        pltpu.SemaphoreType.DMA,
        pltpu.SemaphoreType.DMA,
        pltpu.SemaphoreType.DMA,
    ],
)
def _hist_kernel(p_hbm, l_hbm, z_hbm, out_hbm,
                 pbuf0, pbuf1, lbuf0, lbuf1, hist, idx, shared,
                 sp0, sp1, sl0, sl1):
    cid = lax.axis_index("c")
    sid = lax.axis_index("s")
    wid = sid * NC + cid

    pbufs = (pbuf0, pbuf1)
    lbufs = (lbuf0, lbuf1)
    psems = (sp0, sp1)
    lsems = (sl0, sl1)

    def piece_coords(i):
        pi = wid * PPW + i
        b = pi // 304
        r1 = pi % 304
        c = r1 // 16
        r2 = r1 % 16
        t = r2 // 4
        q = r2 % 4
        return b, c, t, q

    def issue(i):
        b, c, t, q = piece_coords(i)
        rows = pl.ds(q * (PIECE // 256), PIECE // 256)
        slot = i % 2
        hp = pltpu.async_copy(
            p_hbm.at[b, c, t, rows, :], pbufs[slot], psems[slot])
        hl = pltpu.async_copy(
            l_hbm.at[b, c, t, rows, :], lbufs[slot], lsems[slot])
        return hp, hl

    handles = issue(0)
    pltpu.sync_copy(
        z_hbm.at[pl.ds(sid * _STRIPE, _STRIPE)],
        shared.at[pl.ds(sid * _STRIPE, _STRIPE)],
    )
    pltpu.sync_copy(z_hbm.at[pl.ds(0, HROWS)], hist)
    plsc.subcore_barrier()

    ones = jnp.full((16,), 1.0, jnp.float32)
    it16 = lax.iota(jnp.int32, 16)

    def flush(c):
        base = c * HROWS
        for jj in range(HROWS // 16):
            idx[pl.ds(jj * 16, 16)] = base + jj * 16 + it16
        pltpu.sync_copy(hist, shared.at[idx], add=True)
        pltpu.sync_copy(z_hbm.at[pl.ds(0, HROWS)], hist)

    cur_c = piece_coords(0)[1]
    for i in range(PPW):
        if i + 1 < PPW:
            next_handles = issue(i + 1)
        handles[0].wait()
        handles[1].wait()
        if i > 0:
            c_i = piece_coords(i)[1]

            @pl.when(c_i != cur_c)
            def _():
                flush(cur_c)

            cur_c = c_i
        pbuf = pbufs[i % 2]
        lbuf = lbufs[i % 2]

        def vec_body(v, pbuf=pbuf, lbuf=lbuf):
            r = lax.shift_right_logical(v, 4)
            l0 = lax.shift_left(lax.bitwise_and(v, 15), 4)
            p = pbuf[r, pl.ds(l0, 16)]
            lbl = lbuf[r, pl.ds(l0, 16)]
            fgm = lbl > 0
            e = jnp.where(fgm, 1.0 - p, p)
            ki = (e * KS).astype(jnp.int32)
            k2 = ki + ki + lbl
            row = lax.shift_right_logical(k2, 7)
            col = lax.bitwise_and(k2, 127)
            plsc.addupdate_scatter(hist, [row, col], ones)

        plsc.parallel_loop(0, PIECE // 16, unroll=8)(vec_body)
        if i + 1 < PPW:
            handles = next_handles
    flush(cur_c)
    plsc.subcore_barrier()
    pltpu.sync_copy(
        shared.at[pl.ds(sid * _STRIPE, _STRIPE)],
        out_hbm.at[cid, pl.ds(sid * _STRIPE, _STRIPE)],
    )


def _combine_kernel(h_ref, out_ref):
    h = h_ref[0] + h_ref[1]

    l0 = lax.broadcasted_iota(jnp.int32, (128, 128), 0)
    l1 = lax.broadcasted_iota(jnp.int32, (128, 128), 1)
    A = (l0 >= l1).astype(jnp.float32)
    r0 = lax.broadcasted_iota(jnp.int32, (HROWS, HROWS), 0)
    r1 = lax.broadcasted_iota(jnp.int32, (HROWS, HROWS), 1)
    B = (r1 > r0).astype(jnp.float32)
    lane = lax.broadcasted_iota(jnp.int32, (HROWS, 128), 1)
    odd = (lane % 2 == 1).astype(jnp.float32)
    even = 1.0 - odd

    total = jnp.float32(0.0)
    for c in range(NCLS):
        x = h[c * HROWS:(c + 1) * HROWS]
        xo = x * odd
        nl = jax.lax.dot(x, A, precision=jax.lax.Precision.HIGHEST)
        sl = jax.lax.dot(xo, A, precision=jax.lax.Precision.HIGHEST)
        rs_n = jnp.sum(x, axis=1, keepdims=True)
        rs_s = jnp.sum(xo, axis=1, keepdims=True)
        rse_n = jax.lax.dot(B, rs_n, precision=jax.lax.Precision.HIGHEST)
        rse_s = jax.lax.dot(B, rs_s, precision=jax.lax.Precision.HIGHEST)
        n = nl + rse_n
        s = sl + rse_s
        G = jnp.sum(xo)
        J = n / jnp.maximum(G + n - s, 1e-30)
        total = total + (jnp.sum(J * even) - 0.5) / KS
    out_ref[0, 0] = total / float(NCLS)


def kernel(probas, labels):
    zeros = jnp.zeros((SROWS, 128), jnp.float32)
    hist = _hist_kernel(probas, labels, zeros)
    out = pl.pallas_call(
        _combine_kernel,
        out_shape=jax.ShapeDtypeStruct((1, 1), jnp.float32),
        out_specs=pl.BlockSpec(memory_space=pltpu.SMEM),
    )(hist)
    return out[0, 0]
